# Initial kernel scaffold; baseline (speedup 1.0000x reference)
import jax
import jax.numpy as jnp
from jax.experimental import pallas as pl


def kernel(x, token_ei, stmt_ei, block_ei, W_tok, b_tok, W_stm, b_stm, W_blk, b_blk, g_t, be_t, g_s, be_s, g_b, be_b, W1, b1, W2, b2):
    raise NotImplementedError("write your pallas kernel here")


# speedup vs baseline: 10.6243x; 10.6243x over previous
import functools

import jax
import jax.numpy as jnp
from jax import lax
from jax.experimental import pallas as pl
from jax.experimental.pallas import tpu as pltpu
from jax.experimental.pallas import tpu_sc as plsc

_N = 10000
_D = 128
_H = 128
_NP = 10240
_NC = 2
_NS = 16
_NW = _NC * _NS
_CH = 128
_EBLK = _NW * _CH


def _sc_mesh():
    return plsc.VectorSubcoreMesh(
        core_axis_name="c", subcore_axis_name="s",
        num_cores=_NC, num_subcores=_NS)


@functools.lru_cache(maxsize=None)
def _make_deg(K):
    @functools.partial(
        pl.kernel,
        mesh=_sc_mesh(),
        out_type=jax.ShapeDtypeStruct((_NW, _NP), jnp.float32),
        compiler_params=pltpu.CompilerParams(needs_layout_passes=False),
        scratch_types=[
            pltpu.VMEM((_CH,), jnp.int32),
            pltpu.VMEM((_NP,), jnp.float32),
        ],
    )
    def deg_k(dst_hbm, out_hbm, dvec, counts):
        c = lax.axis_index("c")
        s = lax.axis_index("s")
        wid = s * _NC + c
        zeros16 = jnp.zeros((16,), jnp.float32)
        ones16 = jnp.ones((16,), jnp.float32)

        def zstep(i, carry):
            counts[pl.ds(i * 16, 16)] = zeros16
            return carry
        lax.fori_loop(0, _NP // 16, zstep, 0)

        def estep(j, carry):
            pltpu.sync_copy(dst_hbm.at[wid * K + j], dvec)

            def inner(kk, c2):
                idx = dvec[pl.ds(kk * 16, 16)]
                plsc.addupdate_scatter(counts, [idx], ones16)
                return c2
            lax.fori_loop(0, _CH // 16, inner, 0)
            return carry
        lax.fori_loop(0, K, estep, 0)
        pltpu.sync_copy(counts, out_hbm.at[wid])

    return deg_k


@functools.lru_cache(maxsize=None)
def _make_gs(K):
    rpt = _NP // _NS

    @functools.partial(
        pl.kernel,
        mesh=_sc_mesh(),
        out_type=jax.ShapeDtypeStruct((_NC, _NP, _H), jnp.float32),
        compiler_params=pltpu.CompilerParams(needs_layout_passes=False),
        scratch_types=[
            pltpu.VMEM((_CH,), jnp.int32),
            pltpu.VMEM((1, _CH), jnp.int32),
            pltpu.VMEM((_CH, _H), jnp.float32),
            pltpu.VMEM_SHARED((_NP, _H), jnp.float32),
            pltpu.SemaphoreType.DMA,
        ],
    )
    def gs_k(y_hbm, src_hbm, dst_hbm, zrows_hbm, out_hbm,
             sidx, didx, rows, accum, sem):
        c = lax.axis_index("c")
        s = lax.axis_index("s")
        wid = s * _NC + c
        pltpu.sync_copy(zrows_hbm, accum.at[pl.ds(s * rpt, rpt)])
        plsc.subcore_barrier()

        def estep(j, carry):
            g = wid * K + j
            pltpu.sync_copy(src_hbm.at[g], sidx)
            pltpu.sync_copy(dst_hbm.at[g], didx.at[0])
            pltpu.async_copy(y_hbm.at[sidx], rows, sem).wait()
            pltpu.sync_copy(rows, accum.at[didx.at[0]], add=True)
            return carry
        lax.fori_loop(0, K, estep, 0)
        plsc.subcore_barrier()
        pltpu.sync_copy(accum.at[pl.ds(s * rpt, rpt)],
                        out_hbm.at[c].at[pl.ds(s * rpt, rpt)])

    return gs_k


def _dis_col(cnt):
    t = cnt[0:8] + cnt[8:16] + cnt[16:24] + cnt[24:32]
    tt = t.T
    return lax.rsqrt(1.0 + jnp.sum(tt, axis=1, keepdims=True))


def _ln(a, g, be):
    mu = jnp.mean(a, axis=-1, keepdims=True)
    va = jnp.mean((a - mu) ** 2, axis=-1, keepdims=True)
    return (a - mu) * lax.rsqrt(va + 1e-5) * g + be


def _tc_pre(xp, W, cnt):
    def body(x_ref, w_ref, c_ref, y_ref):
        dis = _dis_col(c_ref[...])
        y_ref[...] = dis * jnp.dot(x_ref[...], w_ref[...],
                                   preferred_element_type=jnp.float32)
    return pl.pallas_call(
        body, out_shape=jax.ShapeDtypeStruct((_NP, _H), jnp.float32),
    )(xp, W, cnt)


def _tc_mid(p, y, cnt, b, g, be, Wn, cntn):
    def body(p_ref, y_ref, c_ref, b_ref, g_ref, be_ref, w_ref, cn_ref,
             xo_ref, yn_ref):
        dis = _dis_col(c_ref[...])
        ss = p_ref[0] + p_ref[1] + y_ref[...]
        a = jnp.maximum(dis * ss + b_ref[...], 0.0)
        xo = _ln(a, g_ref[...], be_ref[...])
        xo_ref[...] = xo
        disn = _dis_col(cn_ref[...])
        yn_ref[...] = disn * jnp.dot(xo, w_ref[...],
                                     preferred_element_type=jnp.float32)
    return pl.pallas_call(
        body,
        out_shape=(jax.ShapeDtypeStruct((_NP, _H), jnp.float32),
                   jax.ShapeDtypeStruct((_NP, _H), jnp.float32)),
    )(p, y, cnt, b, g, be, Wn, cntn)


def _tc_fin(p, y, cnt, b, g, be, xt, xs_, W1, b1, W2, b2):
    def body(p_ref, y_ref, c_ref, b_ref, g_ref, be_ref, xt_ref, xs_ref,
             w1_ref, b1_ref, w2_ref, b2_ref, o_ref):
        dis = _dis_col(c_ref[...])
        ss = p_ref[0] + p_ref[1] + y_ref[...]
        a = jnp.maximum(dis * ss + b_ref[...], 0.0)
        xb = _ln(a, g_ref[...], be_ref[...])
        h1 = (jnp.dot(xt_ref[...], w1_ref[0:_H],
                      preferred_element_type=jnp.float32)
              + jnp.dot(xs_ref[...], w1_ref[_H:2 * _H],
                        preferred_element_type=jnp.float32)
              + jnp.dot(xb, w1_ref[2 * _H:3 * _H],
                        preferred_element_type=jnp.float32)
              + b1_ref[...])
        h1 = h1 * 0.5 * (1.0 + lax.erf(h1 * 0.7071067811865476))
        o_ref[...] = jnp.dot(h1, w2_ref[...],
                             preferred_element_type=jnp.float32) + b2_ref[...]
    return pl.pallas_call(
        body, out_shape=jax.ShapeDtypeStruct((_NP, _H), jnp.float32),
    )(p, y, cnt, b, g, be, xt, xs_, W1, b1, W2, b2)


def kernel(x, token_ei, stmt_ei, block_ei, W_tok, b_tok, W_stm, b_stm,
           W_blk, b_blk, g_t, be_t, g_s, be_s, g_b, be_b, W1, b1, W2, b2):
    xp = jnp.zeros((_NP, _D), jnp.float32).at[:_N].set(x)
    zrows = jnp.zeros((_NP // _NS, _H), jnp.float32)

    def prep(ei):
        e = ei.shape[1]
        ep = ((e + _EBLK - 1) // _EBLK) * _EBLK
        src = jnp.full((ep,), _N, jnp.int32).at[:e].set(ei[0].astype(jnp.int32))
        dst = jnp.full((ep,), _N, jnp.int32).at[:e].set(ei[1].astype(jnp.int32))
        return (src.reshape(ep // _CH, _CH), dst.reshape(ep // _CH, _CH),
                ep // _EBLK)

    st, dt, Kt = prep(token_ei)
    ss_, ds_, Ks = prep(stmt_ei)
    sb, db, Kb = prep(block_ei)

    cnt_t = _make_deg(Kt)(dt)
    cnt_s = _make_deg(Ks)(ds_)
    cnt_b = _make_deg(Kb)(db)

    r = lambda v: v.reshape(1, -1)

    y_t = _tc_pre(xp, W_tok, cnt_t)
    p_t = _make_gs(Kt)(y_t, st, dt, zrows)
    xt, y_s = _tc_mid(p_t, y_t, cnt_t, r(b_tok), r(g_t), r(be_t), W_stm, cnt_s)
    p_s = _make_gs(Ks)(y_s, ss_, ds_, zrows)
    xs2, y_b = _tc_mid(p_s, y_s, cnt_s, r(b_stm), r(g_s), r(be_s), W_blk, cnt_b)
    p_b = _make_gs(Kb)(y_b, sb, db, zrows)
    out = _tc_fin(p_b, y_b, cnt_b, r(b_blk), r(g_b), r(be_b), xt, xs2,
                  W1, r(b1), W2, r(b2))
    return out[:_N]


# speedup vs baseline: 27.7084x; 2.6080x over previous
import functools

import jax
import jax.numpy as jnp
from jax import lax
from jax.experimental import pallas as pl
from jax.experimental.pallas import tpu as pltpu
from jax.experimental.pallas import tpu_sc as plsc

_N = 10000
_D = 128
_H = 128
_NP = 10240
_NC = 2
_NS = 16
_NW = _NC * _NS
_CH = 128
_EBLK = _NW * _CH


def _sc_mesh():
    return plsc.VectorSubcoreMesh(
        core_axis_name="c", subcore_axis_name="s",
        num_cores=_NC, num_subcores=_NS)


@functools.lru_cache(maxsize=None)
def _make_deg(K):
    @functools.partial(
        pl.kernel,
        mesh=_sc_mesh(),
        out_type=jax.ShapeDtypeStruct((_NW, _NP), jnp.float32),
        compiler_params=pltpu.CompilerParams(needs_layout_passes=False),
        scratch_types=[
            pltpu.VMEM((_CH,), jnp.int32),
            pltpu.VMEM((_NP,), jnp.float32),
        ],
    )
    def deg_k(dst_hbm, out_hbm, dvec, counts):
        c = lax.axis_index("c")
        s = lax.axis_index("s")
        wid = s * _NC + c
        zeros16 = jnp.zeros((16,), jnp.float32)
        ones16 = jnp.ones((16,), jnp.float32)

        def zstep(i, carry):
            counts[pl.ds(i * 16, 16)] = zeros16
            return carry
        lax.fori_loop(0, _NP // 16, zstep, 0)

        def estep(j, carry):
            pltpu.sync_copy(dst_hbm.at[wid * K + j], dvec)

            def inner(kk, c2):
                idx = dvec[pl.ds(kk * 16, 16)]
                plsc.addupdate_scatter(counts, [idx], ones16)
                return c2
            lax.fori_loop(0, _CH // 16, inner, 0)
            return carry
        lax.fori_loop(0, K, estep, 0)
        pltpu.sync_copy(counts, out_hbm.at[wid])

    return deg_k


_NBUF = 2


@functools.lru_cache(maxsize=None)
def _make_gs(K):
    rpt = _NP // _NS

    @functools.partial(
        pl.kernel,
        mesh=_sc_mesh(),
        out_type=jax.ShapeDtypeStruct((_NC, _NP, _H), jnp.float32),
        compiler_params=pltpu.CompilerParams(needs_layout_passes=False),
        scratch_types=[
            pltpu.VMEM((K, _CH), jnp.int32),
            pltpu.VMEM((_NBUF, _CH), jnp.int32),
            pltpu.VMEM((_NBUF, _CH, _H), jnp.float32),
            pltpu.VMEM_SHARED((_NP, _H), jnp.float32),
            pltpu.SemaphoreType.DMA,
            pltpu.SemaphoreType.DMA,
            pltpu.SemaphoreType.DMA,
            pltpu.SemaphoreType.DMA,
        ],
    )
    def gs_k(y_hbm, src_hbm, dst_hbm, zrows_hbm, out_hbm,
             sidx, didx, rows, accum, rs0, rs1, ds0, ds1):
        rsem = (rs0, rs1)
        dsem = (ds0, ds1)
        c = lax.axis_index("c")
        s = lax.axis_index("s")
        wid = s * _NC + c
        base = wid * K
        pltpu.sync_copy(zrows_hbm, accum.at[pl.ds(s * rpt, rpt)])
        pltpu.sync_copy(src_hbm.at[wid], sidx)
        for b in range(_NBUF):
            pltpu.async_copy(dst_hbm.at[base + b], didx.at[b], dsem[b])
            pltpu.async_copy(y_hbm.at[sidx.at[b]], rows.at[b], rsem[b])
        plsc.subcore_barrier()

        def estep(jj, carry):
            for b in range(_NBUF):
                g = jj * _NBUF + b
                pltpu.make_async_copy(
                    dst_hbm.at[base + g], didx.at[b], dsem[b]).wait()
                pltpu.make_async_copy(
                    y_hbm.at[sidx.at[g]], rows.at[b], rsem[b]).wait()
                pltpu.sync_copy(rows.at[b], accum.at[didx.at[b]], add=True)
                gn = g + _NBUF
                gn = jnp.where(gn < K, gn, gn - K)
                pltpu.async_copy(dst_hbm.at[base + gn], didx.at[b], dsem[b])
                pltpu.async_copy(y_hbm.at[sidx.at[gn]], rows.at[b], rsem[b])
            return carry
        lax.fori_loop(0, K // _NBUF, estep, 0)
        for b in range(_NBUF):
            pltpu.make_async_copy(
                dst_hbm.at[base + b], didx.at[b], dsem[b]).wait()
            pltpu.make_async_copy(
                y_hbm.at[sidx.at[b]], rows.at[b], rsem[b]).wait()
        plsc.subcore_barrier()
        pltpu.sync_copy(accum.at[pl.ds(s * rpt, rpt)],
                        out_hbm.at[c].at[pl.ds(s * rpt, rpt)])

    return gs_k


def _dis_col(cnt):
    t = cnt[0:8] + cnt[8:16] + cnt[16:24] + cnt[24:32]
    tt = t.T
    return lax.rsqrt(1.0 + jnp.sum(tt, axis=1, keepdims=True))


def _ln(a, g, be):
    mu = jnp.mean(a, axis=-1, keepdims=True)
    va = jnp.mean((a - mu) ** 2, axis=-1, keepdims=True)
    return (a - mu) * lax.rsqrt(va + 1e-5) * g + be


def _tc_pre(xp, W, cnt):
    def body(x_ref, w_ref, c_ref, y_ref):
        dis = _dis_col(c_ref[...])
        y_ref[...] = dis * jnp.dot(x_ref[...], w_ref[...],
                                   preferred_element_type=jnp.float32)
    return pl.pallas_call(
        body, out_shape=jax.ShapeDtypeStruct((_NP, _H), jnp.float32),
    )(xp, W, cnt)


def _tc_mid(p, y, cnt, b, g, be, Wn, cntn):
    def body(p_ref, y_ref, c_ref, b_ref, g_ref, be_ref, w_ref, cn_ref,
             xo_ref, yn_ref):
        dis = _dis_col(c_ref[...])
        ss = p_ref[0] + p_ref[1] + y_ref[...]
        a = jnp.maximum(dis * ss + b_ref[...], 0.0)
        xo = _ln(a, g_ref[...], be_ref[...])
        xo_ref[...] = xo
        disn = _dis_col(cn_ref[...])
        yn_ref[...] = disn * jnp.dot(xo, w_ref[...],
                                     preferred_element_type=jnp.float32)
    return pl.pallas_call(
        body,
        out_shape=(jax.ShapeDtypeStruct((_NP, _H), jnp.float32),
                   jax.ShapeDtypeStruct((_NP, _H), jnp.float32)),
    )(p, y, cnt, b, g, be, Wn, cntn)


def _tc_fin(p, y, cnt, b, g, be, xt, xs_, W1, b1, W2, b2):
    def body(p_ref, y_ref, c_ref, b_ref, g_ref, be_ref, xt_ref, xs_ref,
             w1_ref, b1_ref, w2_ref, b2_ref, o_ref):
        dis = _dis_col(c_ref[...])
        ss = p_ref[0] + p_ref[1] + y_ref[...]
        a = jnp.maximum(dis * ss + b_ref[...], 0.0)
        xb = _ln(a, g_ref[...], be_ref[...])
        h1 = (jnp.dot(xt_ref[...], w1_ref[0:_H],
                      preferred_element_type=jnp.float32)
              + jnp.dot(xs_ref[...], w1_ref[_H:2 * _H],
                        preferred_element_type=jnp.float32)
              + jnp.dot(xb, w1_ref[2 * _H:3 * _H],
                        preferred_element_type=jnp.float32)
              + b1_ref[...])
        h1 = h1 * 0.5 * (1.0 + lax.erf(h1 * 0.7071067811865476))
        o_ref[...] = jnp.dot(h1, w2_ref[...],
                             preferred_element_type=jnp.float32) + b2_ref[...]
    return pl.pallas_call(
        body, out_shape=jax.ShapeDtypeStruct((_NP, _H), jnp.float32),
    )(p, y, cnt, b, g, be, xt, xs_, W1, b1, W2, b2)


def kernel(x, token_ei, stmt_ei, block_ei, W_tok, b_tok, W_stm, b_stm,
           W_blk, b_blk, g_t, be_t, g_s, be_s, g_b, be_b, W1, b1, W2, b2):
    xp = jnp.zeros((_NP, _D), jnp.float32).at[:_N].set(x)
    zrows = jnp.zeros((_NP // _NS, _H), jnp.float32)

    def prep(ei):
        e = ei.shape[1]
        gran = _NBUF * _EBLK
        ep = ((e + gran - 1) // gran) * gran
        pad = jnp.arange(ep, dtype=jnp.int32) % (_NP - _N) + _N
        src = pad.at[:e].set(ei[0].astype(jnp.int32))
        dst = pad.at[:e].set(ei[1].astype(jnp.int32))
        K = ep // _EBLK
        return (src.reshape(_NW, K, _CH), dst.reshape(ep // _CH, _CH), K)

    st, dt, Kt = prep(token_ei)
    ss_, ds_, Ks = prep(stmt_ei)
    sb, db, Kb = prep(block_ei)

    cnt_t = _make_deg(Kt)(dt)
    cnt_s = _make_deg(Ks)(ds_)
    cnt_b = _make_deg(Kb)(db)

    r = lambda v: v.reshape(1, -1)

    y_t = _tc_pre(xp, W_tok, cnt_t)
    p_t = _make_gs(Kt)(y_t, st, dt, zrows)
    xt, y_s = _tc_mid(p_t, y_t, cnt_t, r(b_tok), r(g_t), r(be_t), W_stm, cnt_s)
    p_s = _make_gs(Ks)(y_s, ss_, ds_, zrows)
    xs2, y_b = _tc_mid(p_s, y_s, cnt_s, r(b_stm), r(g_s), r(be_s), W_blk, cnt_b)
    p_b = _make_gs(Kb)(y_b, sb, db, zrows)
    out = _tc_fin(p_b, y_b, cnt_b, r(b_blk), r(g_b), r(be_b), xt, xs2,
                  W1, r(b1), W2, r(b2))
    return out[:_N]


# speedup vs baseline: 29.7109x; 1.0723x over previous
import functools

import jax
import jax.numpy as jnp
from jax import lax
from jax.experimental import pallas as pl
from jax.experimental.pallas import tpu as pltpu
from jax.experimental.pallas import tpu_sc as plsc

_N = 10000
_D = 128
_H = 128
_NP = 10240
_NC = 2
_NS = 16
_NW = _NC * _NS
_CH = 128
_EBLK = _NW * _CH


def _sc_mesh():
    return plsc.VectorSubcoreMesh(
        core_axis_name="c", subcore_axis_name="s",
        num_cores=_NC, num_subcores=_NS)


@functools.lru_cache(maxsize=None)
def _make_deg(Kt, Ks, Kb):
    Kmax = max(Kt, Ks, Kb)

    @functools.partial(
        pl.kernel,
        mesh=_sc_mesh(),
        out_type=jax.ShapeDtypeStruct((3, _NW, _NP), jnp.float32),
        compiler_params=pltpu.CompilerParams(needs_layout_passes=False),
        scratch_types=[
            pltpu.VMEM((Kmax, _CH), jnp.int32),
            pltpu.VMEM((_NP,), jnp.float32),
        ],
    )
    def deg_k(dt_hbm, ds_hbm, db_hbm, zflat_hbm, out_hbm, dvec, counts):
        c = lax.axis_index("c")
        s = lax.axis_index("s")
        wid = s * _NC + c
        ones16 = jnp.ones((16,), jnp.float32)

        for si, (dh, Kc) in enumerate(((dt_hbm, Kt), (ds_hbm, Ks),
                                       (db_hbm, Kb))):
            pltpu.sync_copy(zflat_hbm, counts)
            pltpu.sync_copy(dh.at[wid], dvec.at[pl.ds(0, Kc)])

            def estep(j, carry):
                def inner(kk, c2):
                    idx = dvec[j, pl.ds(kk * 16, 16)]
                    plsc.addupdate_scatter(counts, [idx], ones16)
                    return c2
                lax.fori_loop(0, _CH // 16, inner, 0, unroll=True)
                return carry
            lax.fori_loop(0, Kc, estep, 0)
            pltpu.sync_copy(counts, out_hbm.at[si].at[wid])

    return deg_k


_NBUF = 2


@functools.lru_cache(maxsize=None)
def _make_gs(K):
    rpt = _NP // _NS

    @functools.partial(
        pl.kernel,
        mesh=_sc_mesh(),
        out_type=jax.ShapeDtypeStruct((_NC, _NP, _H), jnp.float32),
        compiler_params=pltpu.CompilerParams(needs_layout_passes=False),
        scratch_types=[
            pltpu.VMEM((K, _CH), jnp.int32),
            pltpu.VMEM((_NBUF, _CH), jnp.int32),
            pltpu.VMEM((_NBUF, _CH, _H), jnp.float32),
            pltpu.VMEM_SHARED((_NP, _H), jnp.float32),
            pltpu.SemaphoreType.DMA,
            pltpu.SemaphoreType.DMA,
            pltpu.SemaphoreType.DMA,
            pltpu.SemaphoreType.DMA,
        ],
    )
    def gs_k(y_hbm, src_hbm, dst_hbm, zrows_hbm, out_hbm,
             sidx, didx, rows, accum, rs0, rs1, ds0, ds1):
        rsem = (rs0, rs1)
        dsem = (ds0, ds1)
        c = lax.axis_index("c")
        s = lax.axis_index("s")
        wid = s * _NC + c
        base = wid * K
        pltpu.sync_copy(zrows_hbm, accum.at[pl.ds(s * rpt, rpt)])
        pltpu.sync_copy(src_hbm.at[wid], sidx)
        for b in range(_NBUF):
            pltpu.async_copy(dst_hbm.at[base + b], didx.at[b], dsem[b])
            pltpu.async_copy(y_hbm.at[sidx.at[b]], rows.at[b], rsem[b])
        plsc.subcore_barrier()

        def estep(jj, carry):
            for b in range(_NBUF):
                g = jj * _NBUF + b
                pltpu.make_async_copy(
                    dst_hbm.at[base + g], didx.at[b], dsem[b]).wait()
                pltpu.make_async_copy(
                    y_hbm.at[sidx.at[g]], rows.at[b], rsem[b]).wait()
                pltpu.sync_copy(rows.at[b], accum.at[didx.at[b]], add=True)
                gn = g + _NBUF
                gn = jnp.where(gn < K, gn, gn - K)
                pltpu.async_copy(dst_hbm.at[base + gn], didx.at[b], dsem[b])
                pltpu.async_copy(y_hbm.at[sidx.at[gn]], rows.at[b], rsem[b])
            return carry
        lax.fori_loop(0, K // _NBUF, estep, 0)
        for b in range(_NBUF):
            pltpu.make_async_copy(
                dst_hbm.at[base + b], didx.at[b], dsem[b]).wait()
            pltpu.make_async_copy(
                y_hbm.at[sidx.at[b]], rows.at[b], rsem[b]).wait()
        plsc.subcore_barrier()
        pltpu.sync_copy(accum.at[pl.ds(s * rpt, rpt)],
                        out_hbm.at[c].at[pl.ds(s * rpt, rpt)])

    return gs_k


def _dis_col(cnt):
    t = cnt[0:8] + cnt[8:16] + cnt[16:24] + cnt[24:32]
    tt = t.T
    return lax.rsqrt(1.0 + jnp.sum(tt, axis=1, keepdims=True))


def _ln(a, g, be):
    mu = jnp.mean(a, axis=-1, keepdims=True)
    va = jnp.mean((a - mu) ** 2, axis=-1, keepdims=True)
    return (a - mu) * lax.rsqrt(va + 1e-5) * g + be


def _tc_pre(xp, W, cnt):
    def body(x_ref, w_ref, c_ref, y_ref):
        dis = _dis_col(c_ref[...])
        y_ref[...] = dis * jnp.dot(x_ref[...], w_ref[...],
                                   preferred_element_type=jnp.float32)
    return pl.pallas_call(
        body, out_shape=jax.ShapeDtypeStruct((_NP, _H), jnp.float32),
    )(xp, W, cnt)


def _tc_mid(p, y, cnt, b, g, be, Wn, cntn):
    def body(p_ref, y_ref, c_ref, b_ref, g_ref, be_ref, w_ref, cn_ref,
             xo_ref, yn_ref):
        dis = _dis_col(c_ref[...])
        ss = p_ref[0] + p_ref[1] + y_ref[...]
        a = jnp.maximum(dis * ss + b_ref[...], 0.0)
        xo = _ln(a, g_ref[...], be_ref[...])
        xo_ref[...] = xo
        disn = _dis_col(cn_ref[...])
        yn_ref[...] = disn * jnp.dot(xo, w_ref[...],
                                     preferred_element_type=jnp.float32)
    return pl.pallas_call(
        body,
        out_shape=(jax.ShapeDtypeStruct((_NP, _H), jnp.float32),
                   jax.ShapeDtypeStruct((_NP, _H), jnp.float32)),
    )(p, y, cnt, b, g, be, Wn, cntn)


def _tc_fin(p, y, cnt, b, g, be, xt, xs_, W1, b1, W2, b2):
    def body(p_ref, y_ref, c_ref, b_ref, g_ref, be_ref, xt_ref, xs_ref,
             w1_ref, b1_ref, w2_ref, b2_ref, o_ref):
        dis = _dis_col(c_ref[...])
        ss = p_ref[0] + p_ref[1] + y_ref[...]
        a = jnp.maximum(dis * ss + b_ref[...], 0.0)
        xb = _ln(a, g_ref[...], be_ref[...])
        h1 = (jnp.dot(xt_ref[...], w1_ref[0:_H],
                      preferred_element_type=jnp.float32)
              + jnp.dot(xs_ref[...], w1_ref[_H:2 * _H],
                        preferred_element_type=jnp.float32)
              + jnp.dot(xb, w1_ref[2 * _H:3 * _H],
                        preferred_element_type=jnp.float32)
              + b1_ref[...])
        h1 = h1 * 0.5 * (1.0 + lax.erf(h1 * 0.7071067811865476))
        o_ref[...] = jnp.dot(h1, w2_ref[...],
                             preferred_element_type=jnp.float32) + b2_ref[...]
    return pl.pallas_call(
        body, out_shape=jax.ShapeDtypeStruct((_NP, _H), jnp.float32),
    )(p, y, cnt, b, g, be, xt, xs_, W1, b1, W2, b2)


def kernel(x, token_ei, stmt_ei, block_ei, W_tok, b_tok, W_stm, b_stm,
           W_blk, b_blk, g_t, be_t, g_s, be_s, g_b, be_b, W1, b1, W2, b2):
    xp = jnp.zeros((_NP, _D), jnp.float32).at[:_N].set(x)
    zrows = jnp.zeros((_NP // _NS, _H), jnp.float32)

    def prep(ei):
        e = ei.shape[1]
        gran = _NBUF * _EBLK
        ep = ((e + gran - 1) // gran) * gran
        pad = jnp.arange(ep, dtype=jnp.int32) % (_NP - _N) + _N
        src = pad.at[:e].set(ei[0].astype(jnp.int32))
        dst = pad.at[:e].set(ei[1].astype(jnp.int32))
        K = ep // _EBLK
        return (src.reshape(_NW, K, _CH), dst.reshape(ep // _CH, _CH),
                dst.reshape(_NW, K, _CH), K)

    st, dt, dt3, Kt = prep(token_ei)
    ss_, ds_, ds3, Ks = prep(stmt_ei)
    sb, db, db3, Kb = prep(block_ei)

    zflat = jnp.zeros((_NP,), jnp.float32)
    cnt = _make_deg(Kt, Ks, Kb)(dt3, ds3, db3, zflat)
    cnt_t, cnt_s, cnt_b = cnt[0], cnt[1], cnt[2]

    r = lambda v: v.reshape(1, -1)

    y_t = _tc_pre(xp, W_tok, cnt_t)
    p_t = _make_gs(Kt)(y_t, st, dt, zrows)
    xt, y_s = _tc_mid(p_t, y_t, cnt_t, r(b_tok), r(g_t), r(be_t), W_stm, cnt_s)
    p_s = _make_gs(Ks)(y_s, ss_, ds_, zrows)
    xs2, y_b = _tc_mid(p_s, y_s, cnt_s, r(b_stm), r(g_s), r(be_s), W_blk, cnt_b)
    p_b = _make_gs(Kb)(y_b, sb, db, zrows)
    out = _tc_fin(p_b, y_b, cnt_b, r(b_blk), r(g_b), r(be_b), xt, xs2,
                  W1, r(b1), W2, r(b2))
    return out[:_N]


# speedup vs baseline: 30.7040x; 1.0334x over previous
import functools

import jax
import jax.numpy as jnp
from jax import lax
from jax.experimental import pallas as pl
from jax.experimental.pallas import tpu as pltpu
from jax.experimental.pallas import tpu_sc as plsc

_N = 10000
_D = 128
_H = 128
_NP = 10240
_NC = 2
_NS = 16
_NW = _NC * _NS
_CH = 112
_EBLK = _NW * _CH


def _sc_mesh():
    return plsc.VectorSubcoreMesh(
        core_axis_name="c", subcore_axis_name="s",
        num_cores=_NC, num_subcores=_NS)


@functools.lru_cache(maxsize=None)
def _make_deg(Kt, Ks, Kb):
    Kmax = max(Kt, Ks, Kb)

    @functools.partial(
        pl.kernel,
        mesh=_sc_mesh(),
        out_type=jax.ShapeDtypeStruct((3, _NW, _NP), jnp.float32),
        compiler_params=pltpu.CompilerParams(needs_layout_passes=False),
        scratch_types=[
            pltpu.VMEM((Kmax, _CH), jnp.int32),
            pltpu.VMEM((_NP,), jnp.float32),
        ],
    )
    def deg_k(dt_hbm, ds_hbm, db_hbm, zflat_hbm, out_hbm, dvec, counts):
        c = lax.axis_index("c")
        s = lax.axis_index("s")
        wid = s * _NC + c
        ones16 = jnp.ones((16,), jnp.float32)

        for si, (dh, Kc) in enumerate(((dt_hbm, Kt), (ds_hbm, Ks),
                                       (db_hbm, Kb))):
            pltpu.sync_copy(zflat_hbm, counts)
            pltpu.sync_copy(dh.at[wid], dvec.at[pl.ds(0, Kc)])

            def estep(j, carry):
                def inner(kk, c2):
                    idx = dvec[j, pl.ds(kk * 16, 16)]
                    plsc.addupdate_scatter(counts, [idx], ones16)
                    return c2
                lax.fori_loop(0, _CH // 16, inner, 0, unroll=True)
                return carry
            lax.fori_loop(0, Kc, estep, 0)
            pltpu.sync_copy(counts, out_hbm.at[si].at[wid])

    return deg_k


_NBUF = 3
_NSI = 2 * _NBUF


@functools.lru_cache(maxsize=None)
def _make_gs(K):
    rpt = _NP // _NS

    @functools.partial(
        pl.kernel,
        mesh=_sc_mesh(),
        out_type=jax.ShapeDtypeStruct((_NC, _NP, _H), jnp.float32),
        compiler_params=pltpu.CompilerParams(needs_layout_passes=False),
        scratch_types=[
            pltpu.VMEM((_NSI, _CH), jnp.int32),
            pltpu.VMEM((_NBUF, _CH), jnp.int32),
            pltpu.VMEM((_NBUF, _CH, _H), jnp.float32),
            pltpu.VMEM_SHARED((_NP, _H), jnp.float32),
        ] + [pltpu.SemaphoreType.DMA] * (_NSI + 2 * _NBUF),
    )
    def gs_k(y_hbm, src_hbm, dst_hbm, zrows_hbm, out_hbm,
             sidx, didx, rows, accum, *sems):
        isem = sems[:_NSI]
        dsem = sems[_NSI:_NSI + _NBUF]
        rsem = sems[_NSI + _NBUF:]
        c = lax.axis_index("c")
        s = lax.axis_index("s")
        wid = s * _NC + c
        base = wid * K
        pltpu.sync_copy(zrows_hbm, accum.at[pl.ds(s * rpt, rpt)])
        for q in range(_NSI):
            pltpu.async_copy(src_hbm.at[wid, q], sidx.at[q], isem[q])
        for q in range(_NBUF):
            pltpu.async_copy(dst_hbm.at[base + q], didx.at[q], dsem[q])
        for q in range(_NBUF):
            pltpu.make_async_copy(
                src_hbm.at[wid, q], sidx.at[q], isem[q]).wait()
            pltpu.async_copy(y_hbm.at[sidx.at[q]], rows.at[q], rsem[q])
        plsc.subcore_barrier()

        def estep(jj, carry):
            for v in range(_NSI):
                b = v % _NBUF
                g = jj * _NSI + v
                pltpu.make_async_copy(
                    dst_hbm.at[base + g], didx.at[b], dsem[b]).wait()
                pltpu.make_async_copy(
                    y_hbm.at[sidx.at[v]], rows.at[b], rsem[b]).wait()
                pltpu.sync_copy(rows.at[b], accum.at[didx.at[b]], add=True)
                g3 = g + _NBUF
                g3 = jnp.where(g3 < K, g3, g3 - K)
                g6 = g + _NSI
                g6 = jnp.where(g6 < K, g6, g6 - K)
                pltpu.async_copy(src_hbm.at[wid, g6], sidx.at[v], isem[v])
                pltpu.async_copy(dst_hbm.at[base + g3], didx.at[b], dsem[b])
                v3 = (v + _NBUF) % _NSI
                pltpu.make_async_copy(
                    src_hbm.at[wid, g3], sidx.at[v3], isem[v3]).wait()
                pltpu.async_copy(y_hbm.at[sidx.at[v3]], rows.at[b], rsem[b])
            return carry
        lax.fori_loop(0, K // _NSI, estep, 0)
        for b in range(_NBUF):
            pltpu.make_async_copy(
                dst_hbm.at[base + b], didx.at[b], dsem[b]).wait()
            pltpu.make_async_copy(
                y_hbm.at[sidx.at[b]], rows.at[b], rsem[b]).wait()
        for q in range(_NBUF, _NSI):
            pltpu.make_async_copy(
                src_hbm.at[wid, q], sidx.at[q], isem[q]).wait()
        plsc.subcore_barrier()
        pltpu.sync_copy(accum.at[pl.ds(s * rpt, rpt)],
                        out_hbm.at[c].at[pl.ds(s * rpt, rpt)])

    return gs_k


def _dis_col(cnt):
    t = cnt[0:8] + cnt[8:16] + cnt[16:24] + cnt[24:32]
    tt = t.T
    return lax.rsqrt(1.0 + jnp.sum(tt, axis=1, keepdims=True))


def _ln(a, g, be):
    mu = jnp.mean(a, axis=-1, keepdims=True)
    va = jnp.mean((a - mu) ** 2, axis=-1, keepdims=True)
    return (a - mu) * lax.rsqrt(va + 1e-5) * g + be


def _tc_pre(xp, W, cnt):
    def body(x_ref, w_ref, c_ref, y_ref):
        dis = _dis_col(c_ref[...])
        y_ref[...] = dis * jnp.dot(x_ref[...], w_ref[...],
                                   preferred_element_type=jnp.float32)
    return pl.pallas_call(
        body, out_shape=jax.ShapeDtypeStruct((_NP, _H), jnp.float32),
    )(xp, W, cnt)


def _tc_mid(p, y, cnt, b, g, be, Wn, cntn):
    def body(p_ref, y_ref, c_ref, b_ref, g_ref, be_ref, w_ref, cn_ref,
             xo_ref, yn_ref):
        dis = _dis_col(c_ref[...])
        ss = p_ref[0] + p_ref[1] + y_ref[...]
        a = jnp.maximum(dis * ss + b_ref[...], 0.0)
        xo = _ln(a, g_ref[...], be_ref[...])
        xo_ref[...] = xo
        disn = _dis_col(cn_ref[...])
        yn_ref[...] = disn * jnp.dot(xo, w_ref[...],
                                     preferred_element_type=jnp.float32)
    return pl.pallas_call(
        body,
        out_shape=(jax.ShapeDtypeStruct((_NP, _H), jnp.float32),
                   jax.ShapeDtypeStruct((_NP, _H), jnp.float32)),
    )(p, y, cnt, b, g, be, Wn, cntn)


def _tc_fin(p, y, cnt, b, g, be, xt, xs_, W1, b1, W2, b2):
    def body(p_ref, y_ref, c_ref, b_ref, g_ref, be_ref, xt_ref, xs_ref,
             w1_ref, b1_ref, w2_ref, b2_ref, o_ref):
        dis = _dis_col(c_ref[...])
        ss = p_ref[0] + p_ref[1] + y_ref[...]
        a = jnp.maximum(dis * ss + b_ref[...], 0.0)
        xb = _ln(a, g_ref[...], be_ref[...])
        h1 = (jnp.dot(xt_ref[...], w1_ref[0:_H],
                      preferred_element_type=jnp.float32)
              + jnp.dot(xs_ref[...], w1_ref[_H:2 * _H],
                        preferred_element_type=jnp.float32)
              + jnp.dot(xb, w1_ref[2 * _H:3 * _H],
                        preferred_element_type=jnp.float32)
              + b1_ref[...])
        h1 = h1 * 0.5 * (1.0 + lax.erf(h1 * 0.7071067811865476))
        o_ref[...] = jnp.dot(h1, w2_ref[...],
                             preferred_element_type=jnp.float32) + b2_ref[...]
    return pl.pallas_call(
        body, out_shape=jax.ShapeDtypeStruct((_NP, _H), jnp.float32),
    )(p, y, cnt, b, g, be, xt, xs_, W1, b1, W2, b2)


def kernel(x, token_ei, stmt_ei, block_ei, W_tok, b_tok, W_stm, b_stm,
           W_blk, b_blk, g_t, be_t, g_s, be_s, g_b, be_b, W1, b1, W2, b2):
    xp = jnp.zeros((_NP, _D), jnp.float32).at[:_N].set(x)
    zrows = jnp.zeros((_NP // _NS, _H), jnp.float32)

    def prep(ei):
        e = ei.shape[1]
        gran = _NSI * _EBLK
        ep = ((e + gran - 1) // gran) * gran
        pad = jnp.arange(ep, dtype=jnp.int32) % (_NP - _N) + _N
        src = pad.at[:e].set(ei[0].astype(jnp.int32))
        dst = pad.at[:e].set(ei[1].astype(jnp.int32))
        K = ep // _EBLK
        return (src.reshape(_NW, K, _CH), dst.reshape(ep // _CH, _CH),
                dst.reshape(_NW, K, _CH), K)

    st, dt, dt3, Kt = prep(token_ei)
    ss_, ds_, ds3, Ks = prep(stmt_ei)
    sb, db, db3, Kb = prep(block_ei)

    zflat = jnp.zeros((_NP,), jnp.float32)
    cnt = _make_deg(Kt, Ks, Kb)(dt3, ds3, db3, zflat)
    cnt_t, cnt_s, cnt_b = cnt[0], cnt[1], cnt[2]

    r = lambda v: v.reshape(1, -1)

    y_t = _tc_pre(xp, W_tok, cnt_t)
    p_t = _make_gs(Kt)(y_t, st, dt, zrows)
    xt, y_s = _tc_mid(p_t, y_t, cnt_t, r(b_tok), r(g_t), r(be_t), W_stm, cnt_s)
    p_s = _make_gs(Ks)(y_s, ss_, ds_, zrows)
    xs2, y_b = _tc_mid(p_s, y_s, cnt_s, r(b_stm), r(g_s), r(be_s), W_blk, cnt_b)
    p_b = _make_gs(Kb)(y_b, sb, db, zrows)
    out = _tc_fin(p_b, y_b, cnt_b, r(b_blk), r(g_b), r(be_b), xt, xs2,
                  W1, r(b1), W2, r(b2))
    return out[:_N]


# speedup vs baseline: 30.7265x; 1.0007x over previous
import functools

import jax
import jax.numpy as jnp
from jax import lax
from jax.experimental import pallas as pl
from jax.experimental.pallas import tpu as pltpu
from jax.experimental.pallas import tpu_sc as plsc

_N = 10000
_D = 128
_H = 128
_NP = 10240
_NC = 2
_NS = 16
_NW = _NC * _NS
_CH = 112
_EBLK = _NW * _CH


def _sc_mesh():
    return plsc.VectorSubcoreMesh(
        core_axis_name="c", subcore_axis_name="s",
        num_cores=_NC, num_subcores=_NS)


@functools.lru_cache(maxsize=None)
def _make_deg(Kt, Ks, Kb):
    Kmax = max(Kt, Ks, Kb)

    @functools.partial(
        pl.kernel,
        mesh=_sc_mesh(),
        out_type=jax.ShapeDtypeStruct((3, _NW, _NP), jnp.float32),
        compiler_params=pltpu.CompilerParams(needs_layout_passes=False),
        scratch_types=[
            pltpu.VMEM((Kmax, _CH), jnp.int32),
            pltpu.VMEM((_NP,), jnp.float32),
        ],
    )
    def deg_k(dt_hbm, ds_hbm, db_hbm, zflat_hbm, out_hbm, dvec, counts):
        c = lax.axis_index("c")
        s = lax.axis_index("s")
        wid = s * _NC + c
        ones16 = jnp.ones((16,), jnp.float32)

        for si, (dh, Kc) in enumerate(((dt_hbm, Kt), (ds_hbm, Ks),
                                       (db_hbm, Kb))):
            pltpu.sync_copy(zflat_hbm, counts)
            pltpu.sync_copy(dh.at[wid], dvec.at[pl.ds(0, Kc)])

            def estep(j, carry):
                def inner(kk, c2):
                    idx = dvec[j, pl.ds(kk * 16, 16)]
                    plsc.addupdate_scatter(counts, [idx], ones16)
                    return c2
                lax.fori_loop(0, _CH // 16, inner, 0, unroll=True)
                return carry
            lax.fori_loop(0, Kc, estep, 0)
            pltpu.sync_copy(counts, out_hbm.at[si].at[wid])

    return deg_k


_NBUF = 3
_NSI = 2 * _NBUF


@functools.lru_cache(maxsize=None)
def _make_gs(K):
    rpt = _NP // _NS

    @functools.partial(
        pl.kernel,
        mesh=_sc_mesh(),
        out_type=jax.ShapeDtypeStruct((_NC, _NP, _H), jnp.float32),
        compiler_params=pltpu.CompilerParams(needs_layout_passes=False),
        scratch_types=[
            pltpu.VMEM((_NSI, _CH), jnp.int32),
            pltpu.VMEM((_NBUF, _CH), jnp.int32),
            pltpu.VMEM((_NBUF, _CH, _H), jnp.float32),
            pltpu.VMEM_SHARED((_NP, _H), jnp.float32),
        ] + [pltpu.SemaphoreType.DMA] * (_NSI + 3 * _NBUF),
    )
    def gs_k(y_hbm, src_hbm, dst_hbm, zrows_hbm, out_hbm,
             sidx, didx, rows, accum, *sems):
        isem = sems[:_NSI]
        dsem = sems[_NSI:_NSI + _NBUF]
        rsem = sems[_NSI + _NBUF:_NSI + 2 * _NBUF]
        rsem2 = sems[_NSI + 2 * _NBUF:]
        CA = 64

        def gather2(vv, bb):
            pltpu.async_copy(y_hbm.at[sidx.at[vv, pl.ds(0, CA)]],
                             rows.at[bb, pl.ds(0, CA)], rsem[bb])
            pltpu.async_copy(y_hbm.at[sidx.at[vv, pl.ds(CA, _CH - CA)]],
                             rows.at[bb, pl.ds(CA, _CH - CA)], rsem2[bb])

        def wait2(vv, bb):
            pltpu.make_async_copy(y_hbm.at[sidx.at[vv, pl.ds(0, CA)]],
                                  rows.at[bb, pl.ds(0, CA)], rsem[bb]).wait()
            pltpu.make_async_copy(
                y_hbm.at[sidx.at[vv, pl.ds(CA, _CH - CA)]],
                rows.at[bb, pl.ds(CA, _CH - CA)], rsem2[bb]).wait()
        c = lax.axis_index("c")
        s = lax.axis_index("s")
        wid = s * _NC + c
        base = wid * K
        pltpu.sync_copy(zrows_hbm, accum.at[pl.ds(s * rpt, rpt)])
        for q in range(_NSI):
            pltpu.async_copy(src_hbm.at[wid, q], sidx.at[q], isem[q])
        for q in range(_NBUF):
            pltpu.async_copy(dst_hbm.at[base + q], didx.at[q], dsem[q])
        for q in range(_NBUF):
            pltpu.make_async_copy(
                src_hbm.at[wid, q], sidx.at[q], isem[q]).wait()
            gather2(q, q)
        plsc.subcore_barrier()

        def estep(jj, carry):
            for v in range(_NSI):
                b = v % _NBUF
                g = jj * _NSI + v
                pltpu.make_async_copy(
                    dst_hbm.at[base + g], didx.at[b], dsem[b]).wait()
                wait2(v, b)
                pltpu.sync_copy(rows.at[b], accum.at[didx.at[b]], add=True)
                g3 = g + _NBUF
                g3 = jnp.where(g3 < K, g3, g3 - K)
                g6 = g + _NSI
                g6 = jnp.where(g6 < K, g6, g6 - K)
                pltpu.async_copy(src_hbm.at[wid, g6], sidx.at[v], isem[v])
                pltpu.async_copy(dst_hbm.at[base + g3], didx.at[b], dsem[b])
                v3 = (v + _NBUF) % _NSI
                pltpu.make_async_copy(
                    src_hbm.at[wid, g3], sidx.at[v3], isem[v3]).wait()
                gather2(v3, b)
            return carry
        lax.fori_loop(0, K // _NSI, estep, 0)
        for b in range(_NBUF):
            pltpu.make_async_copy(
                dst_hbm.at[base + b], didx.at[b], dsem[b]).wait()
            wait2(b, b)
        for q in range(_NBUF, _NSI):
            pltpu.make_async_copy(
                src_hbm.at[wid, q], sidx.at[q], isem[q]).wait()
        plsc.subcore_barrier()
        pltpu.sync_copy(accum.at[pl.ds(s * rpt, rpt)],
                        out_hbm.at[c].at[pl.ds(s * rpt, rpt)])

    return gs_k


def _dis_col(cnt):
    t = cnt[0:8] + cnt[8:16] + cnt[16:24] + cnt[24:32]
    tt = t.T
    return lax.rsqrt(1.0 + jnp.sum(tt, axis=1, keepdims=True))


def _ln(a, g, be):
    mu = jnp.mean(a, axis=-1, keepdims=True)
    va = jnp.mean((a - mu) ** 2, axis=-1, keepdims=True)
    return (a - mu) * lax.rsqrt(va + 1e-5) * g + be


def _tc_pre(xp, W, cnt):
    def body(x_ref, w_ref, c_ref, y_ref):
        dis = _dis_col(c_ref[...])
        y_ref[...] = dis * jnp.dot(x_ref[...], w_ref[...],
                                   preferred_element_type=jnp.float32)
    return pl.pallas_call(
        body, out_shape=jax.ShapeDtypeStruct((_NP, _H), jnp.float32),
    )(xp, W, cnt)


def _tc_mid(p, y, cnt, b, g, be, Wn, cntn):
    def body(p_ref, y_ref, c_ref, b_ref, g_ref, be_ref, w_ref, cn_ref,
             xo_ref, yn_ref):
        dis = _dis_col(c_ref[...])
        ss = p_ref[0] + p_ref[1] + y_ref[...]
        a = jnp.maximum(dis * ss + b_ref[...], 0.0)
        xo = _ln(a, g_ref[...], be_ref[...])
        xo_ref[...] = xo
        disn = _dis_col(cn_ref[...])
        yn_ref[...] = disn * jnp.dot(xo, w_ref[...],
                                     preferred_element_type=jnp.float32)
    return pl.pallas_call(
        body,
        out_shape=(jax.ShapeDtypeStruct((_NP, _H), jnp.float32),
                   jax.ShapeDtypeStruct((_NP, _H), jnp.float32)),
    )(p, y, cnt, b, g, be, Wn, cntn)


def _tc_fin(p, y, cnt, b, g, be, xt, xs_, W1, b1, W2, b2):
    def body(p_ref, y_ref, c_ref, b_ref, g_ref, be_ref, xt_ref, xs_ref,
             w1_ref, b1_ref, w2_ref, b2_ref, o_ref):
        dis = _dis_col(c_ref[...])
        ss = p_ref[0] + p_ref[1] + y_ref[...]
        a = jnp.maximum(dis * ss + b_ref[...], 0.0)
        xb = _ln(a, g_ref[...], be_ref[...])
        h1 = (jnp.dot(xt_ref[...], w1_ref[0:_H],
                      preferred_element_type=jnp.float32)
              + jnp.dot(xs_ref[...], w1_ref[_H:2 * _H],
                        preferred_element_type=jnp.float32)
              + jnp.dot(xb, w1_ref[2 * _H:3 * _H],
                        preferred_element_type=jnp.float32)
              + b1_ref[...])
        h1 = h1 * 0.5 * (1.0 + lax.erf(h1 * 0.7071067811865476))
        o_ref[...] = jnp.dot(h1, w2_ref[...],
                             preferred_element_type=jnp.float32) + b2_ref[...]
    return pl.pallas_call(
        body, out_shape=jax.ShapeDtypeStruct((_NP, _H), jnp.float32),
    )(p, y, cnt, b, g, be, xt, xs_, W1, b1, W2, b2)


def kernel(x, token_ei, stmt_ei, block_ei, W_tok, b_tok, W_stm, b_stm,
           W_blk, b_blk, g_t, be_t, g_s, be_s, g_b, be_b, W1, b1, W2, b2):
    xp = jnp.zeros((_NP, _D), jnp.float32).at[:_N].set(x)
    zrows = jnp.zeros((_NP // _NS, _H), jnp.float32)

    def prep(ei):
        e = ei.shape[1]
        gran = _NSI * _EBLK
        ep = ((e + gran - 1) // gran) * gran
        pad = jnp.arange(ep, dtype=jnp.int32) % (_NP - _N) + _N
        src = pad.at[:e].set(ei[0].astype(jnp.int32))
        dst = pad.at[:e].set(ei[1].astype(jnp.int32))
        K = ep // _EBLK
        return (src.reshape(_NW, K, _CH), dst.reshape(ep // _CH, _CH),
                dst.reshape(_NW, K, _CH), K)

    st, dt, dt3, Kt = prep(token_ei)
    ss_, ds_, ds3, Ks = prep(stmt_ei)
    sb, db, db3, Kb = prep(block_ei)

    zflat = jnp.zeros((_NP,), jnp.float32)
    cnt = _make_deg(Kt, Ks, Kb)(dt3, ds3, db3, zflat)
    cnt_t, cnt_s, cnt_b = cnt[0], cnt[1], cnt[2]

    r = lambda v: v.reshape(1, -1)

    y_t = _tc_pre(xp, W_tok, cnt_t)
    p_t = _make_gs(Kt)(y_t, st, dt, zrows)
    xt, y_s = _tc_mid(p_t, y_t, cnt_t, r(b_tok), r(g_t), r(be_t), W_stm, cnt_s)
    p_s = _make_gs(Ks)(y_s, ss_, ds_, zrows)
    xs2, y_b = _tc_mid(p_s, y_s, cnt_s, r(b_stm), r(g_s), r(be_s), W_blk, cnt_b)
    p_b = _make_gs(Kb)(y_b, sb, db, zrows)
    out = _tc_fin(p_b, y_b, cnt_b, r(b_blk), r(g_b), r(be_b), xt, xs2,
                  W1, r(b1), W2, r(b2))
    return out[:_N]
